# baseline (speedup 1.0000x reference)
import functools

import jax
import jax.numpy as jnp
from jax import lax
from jax.experimental import pallas as pl
from jax.experimental.pallas import tpu as pltpu
from jax.experimental.pallas import tpu_sc as plsc

_B, _N, _MAP, _DY, _DX = 4, 512, 4, 8, 6
_D = 256
_TE = 6
_TD = _TE * _DX
_K = _N * _MAP * _DY
_KT = 2048
_NBLK = _K // _KT
_NC, _NS = 2, 16
_ROWS = 64
_NCD = _D // 16
_NI = 32
_RING = 4


def _tc_t_body(x_ref, t2vw_ref, t2vb_ref, vtw_ref, vtb_ref, given_ref, t_ref):
    x = x_ref[0]
    xn = jnp.where(jnp.isnan(x), 0.0, x)
    xrep = jnp.repeat(xn, _TE, axis=1)
    xa = xrep * t2vw_ref[...] + t2vb_ref[...]
    col = lax.broadcasted_iota(jnp.int32, (_N, _TD), 1)
    tv = jnp.where(col % _TE == 0, xa, jnp.sin(xa))
    tt = jnp.dot(tv, vtw_ref[:_TD, :], preferred_element_type=jnp.float32)
    t_ref[0] = tt + vtb_ref[...] + given_ref[1:2, :]


def _sc_body(t_hbm, local_hbm, yperm_hbm, ygperm_hbm, space_hbm, wrow_hbm,
             given_hbm, val_out, space_out, var_out,
             tbuf, lbuf, ybuf, ygbuf, outbuf, rowbuf, varbuf, gbuf,
             idxl0, idxl1, sem_pre, sem_val, sem_space):
    wid = lax.axis_index("s") * _NC + lax.axis_index("c")
    b0 = wid // 16
    p0 = lax.rem(wid, 16)
    b1 = (wid + 32) // 16
    p1 = lax.rem(wid + 32, 16)

    bs = wid // _NBLK
    seg = lax.rem(wid, _NBLK)
    pltpu.sync_copy(space_hbm.at[pl.ds(seg, 1)], rowbuf.at[pl.ds(0, 1)])
    svec = [rowbuf[0, pl.ds(d * 16, 16)] for d in range(_NCD)]
    for r in range(1, _ROWS):
        for d in range(_NCD):
            rowbuf[r, pl.ds(d * 16, 16)] = svec[d]
    vv = jnp.full((16,), seg, jnp.int32)
    for q in range(_KT // 16):
        varbuf[pl.ds(q * 16, 16)] = vv
    sbase = seg * _KT
    for i in range(_KT // _ROWS):
        pltpu.async_copy(rowbuf, space_out.at[bs, pl.ds(sbase + i * _ROWS,
                                                        _ROWS)], sem_space)
    pltpu.async_copy(varbuf, var_out.at[bs, pl.ds(sbase, _KT)], sem_space)

    cps = [pltpu.async_copy(wrow_hbm, gbuf.at[pl.ds(0, 1)], sem_pre),
           pltpu.async_copy(given_hbm, gbuf.at[pl.ds(1, 2)], sem_pre)]
    iv = jnp.arange(16, dtype=jnp.int32) * 16
    for g, (b, p, il) in enumerate(((b0, p0, idxl0), (b1, p1, idxl1))):
        il[pl.ds(0, 16)] = p + iv
        il[pl.ds(16, 16)] = p + 256 + iv
        cps.append(pltpu.async_copy(t_hbm.at[b, pl.ds(32 * p, 32)],
                                    tbuf.at[g], sem_pre))
        cps.append(pltpu.async_copy(local_hbm.at[il], lbuf.at[g], sem_pre))
        cps.append(pltpu.async_copy(yperm_hbm.at[b, p], ybuf.at[g], sem_pre))
        cps.append(pltpu.async_copy(ygperm_hbm.at[b, p], ygbuf.at[g],
                                    sem_pre))
    for cp in cps:
        cp.wait()
    wy = [gbuf[0, pl.ds(d * 16, 16)] for d in range(_NCD)]
    gdiff = [gbuf[1, pl.ds(d * 16, 16)] - gbuf[2, pl.ds(d * 16, 16)]
             for d in range(_NCD)]

    def block_body(i, carry):
        is0 = i < _NI
        g = jnp.where(is0, 0, 1)
        b = jnp.where(is0, b0, b1)
        p = jnp.where(is0, p0, p1)
        iblk = lax.rem(i, _NI)
        par = lax.rem(i, _RING)

        @pl.when(i >= _RING)
        def _drain_one():
            pltpu.make_async_copy(local_hbm.at[pl.ds(0, 32)],
                                  outbuf.at[pl.ds(0, 32)], sem_val).wait()

        rowbase = par * 32
        lrow = [lbuf[g, iblk, pl.ds(d * 16, 16)] for d in range(_NCD)]
        yr = [ybuf[g, iblk, pl.ds(0, 16)], ybuf[g, iblk, pl.ds(16, 16)]]
        ygr = [ygbuf[g, iblk, pl.ds(0, 16)], ygbuf[g, iblk, pl.ds(16, 16)]]
        for j in range(32):
            y_s = yr[j // 16][j % 16]
            yg_s = ygr[j // 16][j % 16]
            yc = jnp.where(y_s == y_s, y_s, 0.0)
            gn = jnp.where(yg_s == yg_s, 0.0, 1.0)
            r = rowbase + j
            for d in range(_NCD):
                t = tbuf[g, j, pl.ds(d * 16, 16)]
                t = t + lrow[d] + yc * wy[d] + gn * gdiff[d]
                outbuf[r, pl.ds(d * 16, 16)] = t
        n = p + 16 * iblk
        pltpu.async_copy(outbuf.at[pl.ds(rowbase, 32)],
                         val_out.at[b, pl.ds(n * 32, 32)], sem_val)
        return carry

    lax.fori_loop(0, 2 * _NI, block_body, 0)

    for _ in range(_RING):
        pltpu.make_async_copy(local_hbm.at[pl.ds(0, 32)],
                              outbuf.at[pl.ds(0, 32)], sem_val).wait()
    for _ in range(_KT // _ROWS):
        pltpu.make_async_copy(local_hbm.at[pl.ds(0, _ROWS)],
                              rowbuf, sem_space).wait()
    pltpu.make_async_copy(var_out.at[0, pl.ds(0, _KT)], varbuf,
                          sem_space).wait()


def kernel(x, y, t2v_w, t2v_b, local_table, vt_w, vt_b, space_table,
           given_table):
    batch = x.shape[0]
    t2vw_f = t2v_w.reshape(1, _TD)
    t2vb_f = t2v_b.reshape(1, _TD)
    vtb_f = vt_b.reshape(1, _D)

    t_tab = pl.pallas_call(
        _tc_t_body,
        grid=(batch,),
        in_specs=[
            pl.BlockSpec((1, _N, _DX), lambda b: (b, 0, 0)),
            pl.BlockSpec((1, _TD), lambda b: (0, 0)),
            pl.BlockSpec((1, _TD), lambda b: (0, 0)),
            pl.BlockSpec((_TD + 1, _D), lambda b: (0, 0)),
            pl.BlockSpec((1, _D), lambda b: (0, 0)),
            pl.BlockSpec((2, _D), lambda b: (0, 0)),
        ],
        out_specs=pl.BlockSpec((1, _N, _D), lambda b: (b, 0, 0)),
        out_shape=jax.ShapeDtypeStruct((batch, _N, _D), jnp.float32),
    )(x, t2vw_f, t2vb_f, vt_w, vtb_f, given_table)

    y3 = y.reshape(batch, _N, _MAP * _DY)
    yperm = y3.reshape(batch, 32, 16, 32).transpose(0, 2, 1, 3)
    yg3 = jnp.transpose(y, (0, 1, 3, 2)).reshape(batch, _N, _MAP * _DY)
    ygperm = yg3.reshape(batch, 32, 16, 32).transpose(0, 2, 1, 3)
    wrow = vt_w[_TD:_TD + 1, :]

    sc_fill = functools.partial(
        pl.kernel,
        out_type=[
            jax.ShapeDtypeStruct((batch, _K, _D), jnp.float32),
            jax.ShapeDtypeStruct((batch, _K, _D), jnp.float32),
            jax.ShapeDtypeStruct((batch, _K), jnp.int32),
        ],
        mesh=plsc.VectorSubcoreMesh(core_axis_name="c", subcore_axis_name="s"),
        scratch_types=[
            pltpu.VMEM((2, 32, _D), jnp.float32),
            pltpu.VMEM((2, 32, _D), jnp.float32),
            pltpu.VMEM((2, 32, 32), jnp.float32),
            pltpu.VMEM((2, 32, 32), jnp.float32),
            pltpu.VMEM((_RING * 32, _D), jnp.float32),
            pltpu.VMEM((_ROWS, _D), jnp.float32),
            pltpu.VMEM((_KT,), jnp.int32),
            pltpu.VMEM((3, _D), jnp.float32),
            pltpu.VMEM((32,), jnp.int32),
            pltpu.VMEM((32,), jnp.int32),
            pltpu.SemaphoreType.DMA,
            pltpu.SemaphoreType.DMA,
            pltpu.SemaphoreType.DMA,
        ],
    )(_sc_body)
    val, space_emb, var_idx = sc_fill(t_tab, local_table, yperm, ygperm,
                                      space_table, wrow, given_table)
    return (val, space_emb, var_idx)
